# baseline (speedup 1.0000x reference)
import functools

import jax
import jax.numpy as jnp
from jax import lax
from jax.experimental import pallas as pl
from jax.experimental.pallas import tpu as pltpu
from jax.experimental.pallas import tpu_sc as plsc

N = 10000
E = 320000
DN = 128
DE = 16
H = 128

NC = 2
NS = 16
NW = NC * NS
LANES = 16

EPW = E // NW
CHUNK = 80
NCHUNK = EPW // CHUNK


def _node_proj_body(xl_ref, xp_ref,
                    wlps_ref, blps_ref, wlpd_ref, blpd_ref,
                    wpls_ref, bpls_ref, wpld_ref, bpld_ref,
                    alp_ref, blp_ref, apl_ref, bpl_ref):
    xl = xl_ref[...]
    xp = xp_ref[...]
    f32 = jnp.float32
    alp_ref[...] = jnp.dot(xl, wlps_ref[...], preferred_element_type=f32) + blps_ref[...]
    blp_ref[...] = jnp.dot(xp, wlpd_ref[...], preferred_element_type=f32) + blpd_ref[...]
    apl_ref[...] = jnp.dot(xl, wpls_ref[...], preferred_element_type=f32) + bpls_ref[...]
    bpl_ref[...] = jnp.dot(xp, wpld_ref[...], preferred_element_type=f32) + bpld_ref[...]


def _node_projections(x_ligand, x_pocket, Wlps, blps, Wlpd, blpd,
                      Wpls, bpls, Wpld, bpld):
    BN = 1000
    grid = (N // BN,)
    full = lambda shape: pl.BlockSpec(shape, lambda i: (0, 0))
    row = lambda shape: pl.BlockSpec(shape, lambda i: (i, 0))
    outs = jax.ShapeDtypeStruct((N, H), jnp.float32)
    return pl.pallas_call(
        _node_proj_body,
        grid=grid,
        in_specs=[row((BN, DN)), row((BN, DN)),
                  full((DN, H)), full((1, H)), full((DN, H)), full((1, H)),
                  full((DN, H)), full((1, H)), full((DN, H)), full((1, H))],
        out_specs=[row((BN, H))] * 4,
        out_shape=[outs] * 4,
    )(x_ligand, x_pocket,
      Wlps, blps.reshape(1, H), Wlpd, blpd.reshape(1, H),
      Wpls, bpls.reshape(1, H), Wpld, bpld.reshape(1, H))


def _edge_proj_body(feat_ref, w_ref, b_ref, out_ref):
    out_ref[...] = (jnp.dot(feat_ref[...], w_ref[...],
                            preferred_element_type=jnp.float32) + b_ref[...])


def _edge_projection(feat, Wf, bf):
    BE = 2560
    grid = (E // BE,)
    return pl.pallas_call(
        _edge_proj_body,
        grid=grid,
        in_specs=[pl.BlockSpec((BE, DE), lambda i: (i, 0)),
                  pl.BlockSpec((DE, H), lambda i: (0, 0)),
                  pl.BlockSpec((1, H), lambda i: (0, 0))],
        out_specs=pl.BlockSpec((BE, H), lambda i: (i, 0)),
        out_shape=jax.ShapeDtypeStruct((E, H), jnp.float32),
    )(feat, Wf, bf.reshape(1, H))


_SC_MESH = plsc.VectorSubcoreMesh(core_axis_name="c", subcore_axis_name="s")

NBUF = 3


NG = H // LANES


@functools.partial(
    pl.kernel,
    mesh=_SC_MESH,
    out_type=jax.ShapeDtypeStruct((NW, NG, LANES), jnp.float32),
    scratch_types=[
        pltpu.VMEM((EPW,), jnp.int32),
        pltpu.VMEM((EPW,), jnp.int32),
        pltpu.VMEM((NBUF, CHUNK, H), jnp.float32),
        pltpu.VMEM((NBUF, CHUNK, H), jnp.float32),
        pltpu.VMEM((NBUF, CHUNK, H), jnp.float32),
        pltpu.VMEM((NG, LANES), jnp.float32),
    ] + [pltpu.SemaphoreType.DMA] * NBUF,
)
def _sc_edge_reduce(a_hbm, b_hbm, ew_hbm, src_hbm, dst_hbm, out_hbm,
                    idx_s_v, idx_d_v, rows_a_v, rows_b_v, ew_v, acc_v,
                    sem0, sem1, sem2):
    sems = (sem0, sem1, sem2)
    wid = lax.axis_index("s") * NC + lax.axis_index("c")
    base = pl.multiple_of(wid * EPW, 8)

    pltpu.sync_copy(src_hbm.at[pl.ds(base, EPW)], idx_s_v)
    pltpu.sync_copy(dst_hbm.at[pl.ds(base, EPW)], idx_d_v)
    for g in range(NG):
        acc_v[g] = jnp.zeros((LANES,), jnp.float32)

    def chunk_dmas(c, b):
        off = pl.multiple_of(c * CHUNK, 8)
        return (
            pltpu.make_async_copy(a_hbm.at[idx_s_v.at[pl.ds(off, CHUNK)]],
                                  rows_a_v.at[b], sems[b]),
            pltpu.make_async_copy(b_hbm.at[idx_d_v.at[pl.ds(off, CHUNK)]],
                                  rows_b_v.at[b], sems[b]),
            pltpu.make_async_copy(ew_hbm.at[pl.ds(base + off, CHUNK)],
                                  ew_v.at[b], sems[b]),
        )

    def start(c, b):
        for d in chunk_dmas(c, b):
            d.start()

    def wait(c, b):
        for d in chunk_dmas(c, b):
            d.wait()

    def compute(b):
        accs = tuple(acc_v[g] for g in range(NG))

        def _row(r, a):
            return tuple(
                a[g] + (rows_a_v[b, r, pl.ds(g * LANES, LANES)]
                        * rows_b_v[b, r, pl.ds(g * LANES, LANES)]
                        * ew_v[b, r, pl.ds(g * LANES, LANES)])
                for g in range(NG))

        accs = plsc.parallel_loop(0, CHUNK, carry=accs, unroll=2)(_row)
        for g in range(NG):
            acc_v[g] = accs[g]

    for b in range(NBUF):
        start(b, b)

    @pl.loop(0, (NCHUNK // NBUF) * NBUF, step=NBUF)
    def _group(c0):
        for b in range(NBUF):
            c = c0 + b
            wait(c, b)
            compute(b)

            @pl.when(c + NBUF < NCHUNK)
            def _():
                start(c + NBUF, b)

    for t in range((NCHUNK // NBUF) * NBUF, NCHUNK):
        b = t % NBUF
        wait(t, b)
        compute(b)

    pltpu.sync_copy(acc_v, out_hbm.at[wid])


def kernel(x_ligand, x_pocket, edge_lp_feat, edge_pl_feat,
           edge_lp_src, edge_lp_dst, edge_pl_src, edge_pl_dst,
           Wlps, blps, Wlpd, blpd, Wlpe, blpe, wlp, blp,
           Wpls, bpls, Wpld, bpld, Wple, bple, wpl, bpl):
    a_lp, b_lp, a_pl, b_pl = _node_projections(
        x_ligand, x_pocket, Wlps, blps, Wlpd, blpd, Wpls, bpls, Wpld, bpld)

    ew_lp = _edge_projection(edge_lp_feat, Wlpe * wlp[:, 0], blpe * wlp[:, 0])
    ew_pl = _edge_projection(edge_pl_feat, Wple * wpl[:, 0], bple * wpl[:, 0])

    tie = (ew_pl[0, 0] * 0.0).astype(jnp.int32)

    part_lp = _sc_edge_reduce(a_lp, b_lp, ew_lp, edge_lp_src + tie, edge_lp_dst)
    part_pl = _sc_edge_reduce(b_pl, a_pl, ew_pl, edge_pl_src, edge_pl_dst)

    logit_lp = (jnp.sum(part_lp) + E * blp[0]).reshape(1, 1)
    logit_pl = (jnp.sum(part_pl) + E * bpl[0]).reshape(1, 1)
    return (logit_lp, logit_pl)


# speedup vs baseline: 1.1060x; 1.1060x over previous
import functools

import jax
import jax.numpy as jnp
from jax import lax
from jax.experimental import pallas as pl
from jax.experimental.pallas import tpu as pltpu
from jax.experimental.pallas import tpu_sc as plsc

N = 10000
E = 320000
DN = 128
DE = 16
H = 128

NC = 2
NS = 16
NW = NC * NS
LANES = 16

EPW = E // NW
CHUNK = 80
NCHUNK = EPW // CHUNK


def _node_proj_body(xl_ref, xp_ref,
                    wlps_ref, blps_ref, wlpd_ref, blpd_ref,
                    wpls_ref, bpls_ref, wpld_ref, bpld_ref,
                    alp_ref, blp_ref, apl_ref, bpl_ref):
    xl = xl_ref[...]
    xp = xp_ref[...]
    f32 = jnp.float32
    alp_ref[...] = jnp.dot(xl, wlps_ref[...], preferred_element_type=f32) + blps_ref[...]
    blp_ref[...] = jnp.dot(xp, wlpd_ref[...], preferred_element_type=f32) + blpd_ref[...]
    apl_ref[...] = jnp.dot(xl, wpls_ref[...], preferred_element_type=f32) + bpls_ref[...]
    bpl_ref[...] = jnp.dot(xp, wpld_ref[...], preferred_element_type=f32) + bpld_ref[...]


def _node_projections(x_ligand, x_pocket, Wlps, blps, Wlpd, blpd,
                      Wpls, bpls, Wpld, bpld):
    BN = 1000
    grid = (N // BN,)
    full = lambda shape: pl.BlockSpec(shape, lambda i: (0, 0))
    row = lambda shape: pl.BlockSpec(shape, lambda i: (i, 0))
    outs = jax.ShapeDtypeStruct((N, H), jnp.float32)
    return pl.pallas_call(
        _node_proj_body,
        grid=grid,
        in_specs=[row((BN, DN)), row((BN, DN)),
                  full((DN, H)), full((1, H)), full((DN, H)), full((1, H)),
                  full((DN, H)), full((1, H)), full((DN, H)), full((1, H))],
        out_specs=[row((BN, H))] * 4,
        out_shape=[outs] * 4,
    )(x_ligand, x_pocket,
      Wlps, blps.reshape(1, H), Wlpd, blpd.reshape(1, H),
      Wpls, bpls.reshape(1, H), Wpld, bpld.reshape(1, H))


def _edge_reduce_body(feat_ref, w_ref, b_ref, m_ref, out_ref):
    @pl.when(pl.program_id(0) == 0)
    def _():
        out_ref[...] = jnp.zeros_like(out_ref)
    ew = (jnp.dot(feat_ref[...], w_ref[...],
                  preferred_element_type=jnp.float32) + b_ref[...])
    out_ref[...] += jnp.sum(ew * m_ref[...])[None, None]


def _edge_reduce(feat, Wf, bf, m):
    BE = 2560
    grid = (E // BE,)
    return pl.pallas_call(
        _edge_reduce_body,
        grid=grid,
        in_specs=[pl.BlockSpec((BE, DE), lambda i: (i, 0)),
                  pl.BlockSpec((DE, H), lambda i: (0, 0)),
                  pl.BlockSpec((1, H), lambda i: (0, 0)),
                  pl.BlockSpec((BE, H), lambda i: (i, 0))],
        out_specs=pl.BlockSpec((1, 1), lambda i: (0, 0)),
        out_shape=jax.ShapeDtypeStruct((1, 1), jnp.float32),
    )(feat, Wf, bf.reshape(1, H), m)


_SC_MESH = plsc.VectorSubcoreMesh(core_axis_name="c", subcore_axis_name="s")

NBUF = 3

NG = H // LANES


@functools.partial(
    pl.kernel,
    mesh=_SC_MESH,
    out_type=jax.ShapeDtypeStruct((E, H), jnp.float32),
    scratch_types=[
        pltpu.VMEM((EPW,), jnp.int32),
        pltpu.VMEM((EPW,), jnp.int32),
        pltpu.VMEM((NBUF, CHUNK, H), jnp.float32),
        pltpu.VMEM((NBUF, CHUNK, H), jnp.float32),
        pltpu.VMEM((NBUF, CHUNK, H), jnp.float32),
    ] + [pltpu.SemaphoreType.DMA] * (2 * NBUF),
)
def _sc_gather_mul(a_hbm, b_hbm, src_hbm, dst_hbm, m_hbm,
                   idx_s_v, idx_d_v, rows_a_v, rows_b_v, m_v,
                   isem0, isem1, isem2, osem0, osem1, osem2):
    isems = (isem0, isem1, isem2)
    osems = (osem0, osem1, osem2)
    wid = lax.axis_index("s") * NC + lax.axis_index("c")
    base = pl.multiple_of(wid * EPW, 8)

    pltpu.sync_copy(src_hbm.at[pl.ds(base, EPW)], idx_s_v)
    pltpu.sync_copy(dst_hbm.at[pl.ds(base, EPW)], idx_d_v)

    def in_dmas(c, b):
        off = pl.multiple_of(c * CHUNK, 8)
        return (
            pltpu.make_async_copy(a_hbm.at[idx_s_v.at[pl.ds(off, CHUNK)]],
                                  rows_a_v.at[b], isems[b]),
            pltpu.make_async_copy(b_hbm.at[idx_d_v.at[pl.ds(off, CHUNK)]],
                                  rows_b_v.at[b], isems[b]),
        )

    def out_dma(c, b):
        off = pl.multiple_of(c * CHUNK, 8)
        return pltpu.make_async_copy(m_v.at[b],
                                     m_hbm.at[pl.ds(base + off, CHUNK)],
                                     osems[b])

    def start_in(c, b):
        for d in in_dmas(c, b):
            d.start()

    def wait_in(c, b):
        for d in in_dmas(c, b):
            d.wait()

    def compute(b):
        def _row(r):
            for g in range(NG):
                m_v[b, r, pl.ds(g * LANES, LANES)] = (
                    rows_a_v[b, r, pl.ds(g * LANES, LANES)]
                    * rows_b_v[b, r, pl.ds(g * LANES, LANES)])

        plsc.parallel_loop(0, CHUNK, unroll=2)(_row)

    def step(c, b):
        wait_in(c, b)

        @pl.when(c >= NBUF)
        def _():
            out_dma(c - NBUF, b).wait()

        compute(b)
        out_dma(c, b).start()

        @pl.when(c + NBUF < NCHUNK)
        def _():
            start_in(c + NBUF, b)

    for b in range(NBUF):
        start_in(b, b)

    @pl.loop(0, (NCHUNK // NBUF) * NBUF, step=NBUF)
    def _group(c0):
        for b in range(NBUF):
            step(c0 + b, b)

    for t in range((NCHUNK // NBUF) * NBUF, NCHUNK):
        step(t, t % NBUF)

    for c in range(NCHUNK - NBUF, NCHUNK):
        out_dma(c, c % NBUF).wait()


def kernel(x_ligand, x_pocket, edge_lp_feat, edge_pl_feat,
           edge_lp_src, edge_lp_dst, edge_pl_src, edge_pl_dst,
           Wlps, blps, Wlpd, blpd, Wlpe, blpe, wlp, blp,
           Wpls, bpls, Wpld, bpld, Wple, bple, wpl, bpl):
    a_lp, b_lp, a_pl, b_pl = _node_projections(
        x_ligand, x_pocket, Wlps, blps, Wlpd, blpd, Wpls, bpls, Wpld, bpld)

    m_lp = _sc_gather_mul(a_lp, b_lp, edge_lp_src, edge_lp_dst)
    m_pl = _sc_gather_mul(b_pl, a_pl, edge_pl_src, edge_pl_dst)

    s_lp = _edge_reduce(edge_lp_feat, Wlpe * wlp[:, 0], blpe * wlp[:, 0], m_lp)
    s_pl = _edge_reduce(edge_pl_feat, Wple * wpl[:, 0], bple * wpl[:, 0], m_pl)

    logit_lp = s_lp + E * blp[0]
    logit_pl = s_pl + E * bpl[0]
    return (logit_lp, logit_pl)


# speedup vs baseline: 1.1953x; 1.0807x over previous
import functools

import jax
import jax.numpy as jnp
from jax import lax
from jax.experimental import pallas as pl
from jax.experimental.pallas import tpu as pltpu
from jax.experimental.pallas import tpu_sc as plsc

N = 10000
E = 320000
DN = 128
DE = 16
H = 128

NC = 2
NS = 16
NW = NC * NS
LANES = 16

EPW = E // NW
CHUNK = 80
NCHUNK = EPW // CHUNK


def _node_proj_body(xl_ref, xp_ref,
                    wlps_ref, blps_ref, wlpd_ref, blpd_ref,
                    wpls_ref, bpls_ref, wpld_ref, bpld_ref,
                    alp_ref, blp_ref, apl_ref, bpl_ref):
    xl = xl_ref[...]
    xp = xp_ref[...]
    f32 = jnp.float32
    alp_ref[...] = jnp.dot(xl, wlps_ref[...], preferred_element_type=f32) + blps_ref[...]
    blp_ref[...] = jnp.dot(xp, wlpd_ref[...], preferred_element_type=f32) + blpd_ref[...]
    apl_ref[...] = jnp.dot(xl, wpls_ref[...], preferred_element_type=f32) + bpls_ref[...]
    bpl_ref[...] = jnp.dot(xp, wpld_ref[...], preferred_element_type=f32) + bpld_ref[...]


def _node_projections(x_ligand, x_pocket, Wlps, blps, Wlpd, blpd,
                      Wpls, bpls, Wpld, bpld):
    BN = 1000
    grid = (N // BN,)
    full = lambda shape: pl.BlockSpec(shape, lambda i: (0, 0))
    row = lambda shape: pl.BlockSpec(shape, lambda i: (i, 0))
    outs = jax.ShapeDtypeStruct((N, H), jnp.float32)
    return pl.pallas_call(
        _node_proj_body,
        grid=grid,
        in_specs=[row((BN, DN)), row((BN, DN)),
                  full((DN, H)), full((1, H)), full((DN, H)), full((1, H)),
                  full((DN, H)), full((1, H)), full((DN, H)), full((1, H))],
        out_specs=[row((BN, H))] * 4,
        out_shape=[outs] * 4,
    )(x_ligand, x_pocket,
      Wlps, blps.reshape(1, H), Wlpd, blpd.reshape(1, H),
      Wpls, bpls.reshape(1, H), Wpld, bpld.reshape(1, H))


def _edge_reduce_body(feat_ref, w_ref, b_ref, m_ref, out_ref):
    @pl.when(pl.program_id(0) == 0)
    def _():
        out_ref[...] = jnp.zeros_like(out_ref)
    ew = (jnp.dot(feat_ref[...], w_ref[...],
                  preferred_element_type=jnp.float32) + b_ref[...])
    out_ref[...] += jnp.sum(ew * m_ref[...])[None, None]


def _edge_reduce(feat, Wf, bf, m):
    BE = 6400
    grid = (E // BE,)
    return pl.pallas_call(
        _edge_reduce_body,
        grid=grid,
        in_specs=[pl.BlockSpec((BE, DE), lambda i: (i, 0)),
                  pl.BlockSpec((DE, H), lambda i: (0, 0)),
                  pl.BlockSpec((1, H), lambda i: (0, 0)),
                  pl.BlockSpec((BE, H), lambda i: (i, 0))],
        out_specs=pl.BlockSpec((1, 1), lambda i: (0, 0)),
        out_shape=jax.ShapeDtypeStruct((1, 1), jnp.float32),
    )(feat, Wf, bf.reshape(1, H), m)


_SC_MESH = plsc.VectorSubcoreMesh(core_axis_name="c", subcore_axis_name="s")

NBUF = 3

NG = H // LANES


@functools.partial(
    pl.kernel,
    mesh=_SC_MESH,
    out_type=jax.ShapeDtypeStruct((E, H), jnp.float32),
    scratch_types=[
        pltpu.VMEM((EPW,), jnp.int32),
        pltpu.VMEM((EPW,), jnp.int32),
        pltpu.VMEM((NBUF, CHUNK, H), jnp.float32),
        pltpu.VMEM((NBUF, CHUNK, H), jnp.float32),
        pltpu.VMEM((NBUF, CHUNK, H), jnp.float32),
    ] + [pltpu.SemaphoreType.DMA] * (2 * NBUF),
)
def _sc_gather_mul(a_hbm, b_hbm, src_hbm, dst_hbm, m_hbm,
                   idx_s_v, idx_d_v, rows_a_v, rows_b_v, m_v,
                   isem0, isem1, isem2, osem0, osem1, osem2):
    isems = (isem0, isem1, isem2)
    osems = (osem0, osem1, osem2)
    wid = lax.axis_index("s") * NC + lax.axis_index("c")
    base = pl.multiple_of(wid * EPW, 8)

    pltpu.sync_copy(src_hbm.at[pl.ds(base, EPW)], idx_s_v)
    pltpu.sync_copy(dst_hbm.at[pl.ds(base, EPW)], idx_d_v)

    def in_dmas(c, b):
        off = pl.multiple_of(c * CHUNK, 8)
        return (
            pltpu.make_async_copy(a_hbm.at[idx_s_v.at[pl.ds(off, CHUNK)]],
                                  rows_a_v.at[b], isems[b]),
            pltpu.make_async_copy(b_hbm.at[idx_d_v.at[pl.ds(off, CHUNK)]],
                                  rows_b_v.at[b], isems[b]),
        )

    def out_dma(c, b):
        off = pl.multiple_of(c * CHUNK, 8)
        return pltpu.make_async_copy(m_v.at[b],
                                     m_hbm.at[pl.ds(base + off, CHUNK)],
                                     osems[b])

    def start_in(c, b):
        for d in in_dmas(c, b):
            d.start()

    def wait_in(c, b):
        for d in in_dmas(c, b):
            d.wait()

    def compute(b):
        def _row(r):
            for g in range(NG):
                m_v[b, r, pl.ds(g * LANES, LANES)] = (
                    rows_a_v[b, r, pl.ds(g * LANES, LANES)]
                    * rows_b_v[b, r, pl.ds(g * LANES, LANES)])

        plsc.parallel_loop(0, CHUNK, unroll=4)(_row)

    def step(c, b):
        wait_in(c, b)

        @pl.when(c >= NBUF)
        def _():
            out_dma(c - NBUF, b).wait()

        compute(b)
        out_dma(c, b).start()

        @pl.when(c + NBUF < NCHUNK)
        def _():
            start_in(c + NBUF, b)

    for b in range(NBUF):
        start_in(b, b)

    @pl.loop(0, (NCHUNK // NBUF) * NBUF, step=NBUF)
    def _group(c0):
        for b in range(NBUF):
            step(c0 + b, b)

    for t in range((NCHUNK // NBUF) * NBUF, NCHUNK):
        step(t, t % NBUF)

    for c in range(NCHUNK - NBUF, NCHUNK):
        out_dma(c, c % NBUF).wait()


def kernel(x_ligand, x_pocket, edge_lp_feat, edge_pl_feat,
           edge_lp_src, edge_lp_dst, edge_pl_src, edge_pl_dst,
           Wlps, blps, Wlpd, blpd, Wlpe, blpe, wlp, blp,
           Wpls, bpls, Wpld, bpld, Wple, bple, wpl, bpl):
    a_lp, b_lp, a_pl, b_pl = _node_projections(
        x_ligand, x_pocket, Wlps, blps, Wlpd, blpd, Wpls, bpls, Wpld, bpld)

    m_lp = _sc_gather_mul(a_lp, b_lp, edge_lp_src, edge_lp_dst)
    m_pl = _sc_gather_mul(b_pl, a_pl, edge_pl_src, edge_pl_dst)

    s_lp = _edge_reduce(edge_lp_feat, Wlpe * wlp[:, 0], blpe * wlp[:, 0], m_lp)
    s_pl = _edge_reduce(edge_pl_feat, Wple * wpl[:, 0], bple * wpl[:, 0], m_pl)

    logit_lp = s_lp + E * blp[0]
    logit_pl = s_pl + E * bpl[0]
    return (logit_lp, logit_pl)


# speedup vs baseline: 1.2050x; 1.0080x over previous
import functools

import jax
import jax.numpy as jnp
from jax import lax
from jax.experimental import pallas as pl
from jax.experimental.pallas import tpu as pltpu
from jax.experimental.pallas import tpu_sc as plsc

N = 10000
E = 320000
DN = 128
DE = 16
H = 128

NC = 2
NS = 16
NW = NC * NS
LANES = 16

EPW = E // NW
CHUNK = 80
NCHUNK = EPW // CHUNK


def _node_proj_body(xl_ref, xp_ref,
                    wlps_ref, blps_ref, wlpd_ref, blpd_ref,
                    wpls_ref, bpls_ref, wpld_ref, bpld_ref,
                    alp_ref, blp_ref, apl_ref, bpl_ref):
    xl = xl_ref[...]
    xp = xp_ref[...]
    f32 = jnp.float32
    alp_ref[...] = jnp.dot(xl, wlps_ref[...], preferred_element_type=f32) + blps_ref[...]
    blp_ref[...] = jnp.dot(xp, wlpd_ref[...], preferred_element_type=f32) + blpd_ref[...]
    apl_ref[...] = jnp.dot(xl, wpls_ref[...], preferred_element_type=f32) + bpls_ref[...]
    bpl_ref[...] = jnp.dot(xp, wpld_ref[...], preferred_element_type=f32) + bpld_ref[...]


def _node_projections(x_ligand, x_pocket, Wlps, blps, Wlpd, blpd,
                      Wpls, bpls, Wpld, bpld):
    BN = 1000
    grid = (N // BN,)
    full = lambda shape: pl.BlockSpec(shape, lambda i: (0, 0))
    row = lambda shape: pl.BlockSpec(shape, lambda i: (i, 0))
    outs = jax.ShapeDtypeStruct((N, H), jnp.float32)
    return pl.pallas_call(
        _node_proj_body,
        grid=grid,
        in_specs=[row((BN, DN)), row((BN, DN)),
                  full((DN, H)), full((1, H)), full((DN, H)), full((1, H)),
                  full((DN, H)), full((1, H)), full((DN, H)), full((1, H))],
        out_specs=[row((BN, H))] * 4,
        out_shape=[outs] * 4,
    )(x_ligand, x_pocket,
      Wlps, blps.reshape(1, H), Wlpd, blpd.reshape(1, H),
      Wpls, bpls.reshape(1, H), Wpld, bpld.reshape(1, H))


def _edge_reduce_body(feat_ref, w_ref, b_ref, m_ref, out_ref):
    @pl.when(pl.program_id(0) == 0)
    def _():
        out_ref[...] = jnp.zeros_like(out_ref)
    ew = (jnp.dot(feat_ref[...], w_ref[...],
                  preferred_element_type=jnp.float32) + b_ref[...])
    out_ref[...] += jnp.sum(ew * m_ref[...])[None, None]


def _edge_reduce(feat, Wf, bf, m):
    BE = 8000
    grid = (E // BE,)
    return pl.pallas_call(
        _edge_reduce_body,
        grid=grid,
        in_specs=[pl.BlockSpec((BE, DE), lambda i: (i, 0)),
                  pl.BlockSpec((DE, H), lambda i: (0, 0)),
                  pl.BlockSpec((1, H), lambda i: (0, 0)),
                  pl.BlockSpec((BE, H), lambda i: (i, 0))],
        out_specs=pl.BlockSpec((1, 1), lambda i: (0, 0)),
        out_shape=jax.ShapeDtypeStruct((1, 1), jnp.float32),
    )(feat, Wf, bf.reshape(1, H), m)


_SC_MESH = plsc.VectorSubcoreMesh(core_axis_name="c", subcore_axis_name="s")

NBUF = 3

NG = H // LANES


@functools.partial(
    pl.kernel,
    mesh=_SC_MESH,
    out_type=jax.ShapeDtypeStruct((E, H), jnp.float32),
    scratch_types=[
        pltpu.VMEM((EPW,), jnp.int32),
        pltpu.VMEM((EPW,), jnp.int32),
        pltpu.VMEM((NBUF, CHUNK, H), jnp.float32),
        pltpu.VMEM((NBUF, CHUNK, H), jnp.float32),
        pltpu.VMEM((NBUF, CHUNK, H), jnp.float32),
    ] + [pltpu.SemaphoreType.DMA] * (2 * NBUF),
)
def _sc_gather_mul(a_hbm, b_hbm, src_hbm, dst_hbm, m_hbm,
                   idx_s_v, idx_d_v, rows_a_v, rows_b_v, m_v,
                   isem0, isem1, isem2, osem0, osem1, osem2):
    isems = (isem0, isem1, isem2)
    osems = (osem0, osem1, osem2)
    wid = lax.axis_index("s") * NC + lax.axis_index("c")
    base = pl.multiple_of(wid * EPW, 8)

    pltpu.sync_copy(src_hbm.at[pl.ds(base, EPW)], idx_s_v)
    pltpu.sync_copy(dst_hbm.at[pl.ds(base, EPW)], idx_d_v)

    def in_dmas(c, b):
        off = pl.multiple_of(c * CHUNK, 8)
        return (
            pltpu.make_async_copy(a_hbm.at[idx_s_v.at[pl.ds(off, CHUNK)]],
                                  rows_a_v.at[b], isems[b]),
            pltpu.make_async_copy(b_hbm.at[idx_d_v.at[pl.ds(off, CHUNK)]],
                                  rows_b_v.at[b], isems[b]),
        )

    def out_dma(c, b):
        off = pl.multiple_of(c * CHUNK, 8)
        return pltpu.make_async_copy(m_v.at[b],
                                     m_hbm.at[pl.ds(base + off, CHUNK)],
                                     osems[b])

    def start_in(c, b):
        for d in in_dmas(c, b):
            d.start()

    def wait_in(c, b):
        for d in in_dmas(c, b):
            d.wait()

    def compute(b):
        def _row(r):
            for g in range(NG):
                m_v[b, r, pl.ds(g * LANES, LANES)] = (
                    rows_a_v[b, r, pl.ds(g * LANES, LANES)]
                    * rows_b_v[b, r, pl.ds(g * LANES, LANES)])

        plsc.parallel_loop(0, CHUNK, unroll=8)(_row)

    def step(c, b):
        wait_in(c, b)

        @pl.when(c >= NBUF)
        def _():
            out_dma(c - NBUF, b).wait()

        compute(b)
        out_dma(c, b).start()

        @pl.when(c + NBUF < NCHUNK)
        def _():
            start_in(c + NBUF, b)

    for b in range(NBUF):
        start_in(b, b)

    @pl.loop(0, (NCHUNK // NBUF) * NBUF, step=NBUF)
    def _group(c0):
        for b in range(NBUF):
            step(c0 + b, b)

    for t in range((NCHUNK // NBUF) * NBUF, NCHUNK):
        step(t, t % NBUF)

    for c in range(NCHUNK - NBUF, NCHUNK):
        out_dma(c, c % NBUF).wait()


def kernel(x_ligand, x_pocket, edge_lp_feat, edge_pl_feat,
           edge_lp_src, edge_lp_dst, edge_pl_src, edge_pl_dst,
           Wlps, blps, Wlpd, blpd, Wlpe, blpe, wlp, blp,
           Wpls, bpls, Wpld, bpld, Wple, bple, wpl, bpl):
    a_lp, b_lp, a_pl, b_pl = _node_projections(
        x_ligand, x_pocket, Wlps, blps, Wlpd, blpd, Wpls, bpls, Wpld, bpld)

    m_lp = _sc_gather_mul(a_lp, b_lp, edge_lp_src, edge_lp_dst)
    m_pl = _sc_gather_mul(b_pl, a_pl, edge_pl_src, edge_pl_dst)

    s_lp = _edge_reduce(edge_lp_feat, Wlpe * wlp[:, 0], blpe * wlp[:, 0], m_lp)
    s_pl = _edge_reduce(edge_pl_feat, Wple * wpl[:, 0], bple * wpl[:, 0], m_pl)

    logit_lp = s_lp + E * blp[0]
    logit_pl = s_pl + E * bpl[0]
    return (logit_lp, logit_pl)
